# Initial kernel scaffold; baseline (speedup 1.0000x reference)
import jax
import jax.numpy as jnp
from jax.experimental import pallas as pl


def kernel(x, pos, batch, params):
    raise NotImplementedError("write your pallas kernel here")


# speedup vs baseline: 1.2014x; 1.2014x over previous
import functools

import jax
import jax.numpy as jnp
import numpy as np
from jax.experimental import pallas as pl

_B = 8
_N0 = 512
_M1 = int(np.ceil(_N0 * 0.999))
_M2 = int(np.ceil(_M1 * 0.33))
_R1 = 0.4
_R2 = 0.6
_KNBR = 64


def _mlp(params, x):
    for W, b in params:
        x = jnp.maximum(jnp.dot(x, W) + b, 0.0)
    return x


def _fps(pts, m):
    n = pts.shape[0]
    idxs0 = jnp.zeros((m,), dtype=jnp.int32)
    d0 = jnp.full((n,), jnp.inf, dtype=jnp.float32)

    def body(i, st):
        idxs, d = st
        last = pts[idxs[i - 1]]
        dd = jnp.sum((pts - last) ** 2, axis=1)
        d = jnp.minimum(d, dd)
        idxs = idxs.at[i].set(jnp.argmax(d).astype(jnp.int32))
        return (idxs, d)

    idxs, _ = jax.lax.fori_loop(1, m, body, (idxs0, d0))
    return idxs


def _point_conv(x, pos, cpos, r, params):
    d2 = jnp.sum((cpos[:, None, :] - pos[None, :, :]) ** 2, axis=-1)
    neg, nbr = jax.lax.top_k(-d2, _KNBR)
    valid = (-neg) <= r * r
    xj = x[nbr]
    rel = pos[nbr] - cpos[:, None, :]
    msg = _mlp(params, jnp.concatenate([xj, rel], axis=-1))
    msg = jnp.where(valid[:, :, None], msg, -jnp.inf)
    return jnp.max(msg, axis=1)


def _knn_interp(x, pos_x, pos_y, k):
    d2 = jnp.sum((pos_y[:, None, :] - pos_x[None, :, :]) ** 2, axis=-1)
    neg, idx = jax.lax.top_k(-d2, k)
    w = 1.0 / jnp.clip(-neg, 1e-16, None)
    xf = x[idx]
    return jnp.sum(xf * w[:, :, None], axis=1) / jnp.sum(w, axis=1, keepdims=True)


def _fp1_body(h_ref, w1_ref, b1_ref, w2_ref, b2_ref, o_ref):
    h = jnp.maximum(jnp.dot(h_ref[...], w1_ref[...],
                            preferred_element_type=jnp.float32) + b1_ref[...], 0.0)
    o_ref[...] = jnp.maximum(jnp.dot(h, w2_ref[...],
                                     preferred_element_type=jnp.float32) + b2_ref[...], 0.0)


def kernel(x, pos, batch, params):
    xb = x.reshape(_B, _N0, -1)
    pb = pos.reshape(_B, _N0, 3)
    x1 = jax.vmap(lambda xc, pc: _point_conv(xc, pc, pc, _R1, params['sa1']))(xb, pb)
    p1 = pb
    idx2 = jax.vmap(lambda p: _fps(p, _M2))(p1)
    p2 = jax.vmap(lambda pc, ic: pc[ic])(p1, idx2)
    x2 = jax.vmap(lambda xc, pc, cp: _point_conv(xc, pc, cp, _R2, params['sa2']))(x1, p1, p2)
    h3 = _mlp(params['sa3'], jnp.concatenate([x2, p2], axis=-1))
    x3 = jnp.max(h3, axis=1, keepdims=True)
    xi3 = jnp.broadcast_to(x3, (_B, _M2, x3.shape[-1]))
    f3 = _mlp(params['fp3'], jnp.concatenate([xi3, x2], axis=-1))
    xi2 = jax.vmap(lambda xc, px, py: _knn_interp(xc, px, py, 3))(f3, p2, p1)
    f2 = _mlp(params['fp2'], jnp.concatenate([xi2, x1], axis=-1))
    xi1 = jax.vmap(lambda xc, px, py: _knn_interp(xc, px, py, 3))(f2, p1, pb)
    h = jnp.concatenate([xi1, xb], axis=-1).reshape(_B * _N0, -1)
    (w1, b1), (w2, b2) = params['fp1']
    hp = jnp.pad(h, ((0, 0), (0, 256 - h.shape[1])))
    w1p = jnp.pad(w1, ((0, 256 - w1.shape[0]), (0, 0)))
    out = pl.pallas_call(
        _fp1_body,
        out_shape=jax.ShapeDtypeStruct((_B * _N0, 128), jnp.float32),
    )(hp, w1p, b1[None, :], w2, b2[None, :])
    return out


# speedup vs baseline: 7.3058x; 6.0812x over previous
import jax
import jax.numpy as jnp
import numpy as np
from jax.experimental import pallas as pl
from jax.experimental.pallas import tpu as pltpu

_B = 8
_N0 = 512
_M1 = int(np.ceil(_N0 * 0.999))
_M2 = int(np.ceil(_M1 * 0.33))
_MP = 192
_R1 = 0.4
_R2 = 0.6
_KNBR = 64
_TC1 = 8
_TC2 = 8


def _search2(bits_fn, rows, k_a, k_b):
    lo_a0 = jnp.full((rows, 1), -1, dtype=jnp.int32)
    hi_a0 = jnp.full((rows, 1), 0x40A00000, dtype=jnp.int32)

    def body(_, st):
        lo_a, hi_a, lo_b, hi_b = st
        bits = bits_fn()
        mid_a = lo_a + jax.lax.shift_right_arithmetic(hi_a - lo_a, 1)
        cnt_a = jnp.sum((bits <= mid_a).astype(jnp.int32), axis=1, keepdims=True)
        lo_a = jnp.where(cnt_a < k_a, mid_a, lo_a)
        hi_a = jnp.where(cnt_a >= k_a, mid_a, hi_a)
        mid_b = lo_b + jax.lax.shift_right_arithmetic(hi_b - lo_b, 1)
        cnt_b = jnp.sum((bits <= mid_b).astype(jnp.int32), axis=1, keepdims=True)
        lo_b = jnp.where(cnt_b < k_b, mid_b, lo_b)
        hi_b = jnp.where(cnt_b >= k_b, mid_b, hi_b)
        return (lo_a, hi_a, lo_b, hi_b)

    st = jax.lax.fori_loop(0, 32, body, (lo_a0, hi_a0, lo_a0, hi_a0))
    return (jax.lax.bitcast_convert_type(st[1], jnp.float32),
            jax.lax.bitcast_convert_type(st[3], jnp.float32))


def _fps_body(pt_ref, out_ref):
    px = pt_ref[0]
    py = pt_ref[1]
    pz = pt_ref[2]
    lane = jax.lax.broadcasted_iota(jnp.int32, (_B, _N0), 1)
    lane2 = jax.lax.broadcasted_iota(jnp.int32, (_B, _M2), 1)

    lx = px[:, 0:1]
    ly = py[:, 0:1]
    lz = pz[:, 0:1]
    oh0 = (lane2 == 0).astype(jnp.float32)
    ax = lx * oh0
    ay = ly * oh0
    az = lz * oh0
    d = jnp.full((_B, _N0), jnp.inf, dtype=jnp.float32)

    def body(i, st):
        d, lx, ly, lz, ax, ay, az = st
        dd = (px - lx) ** 2 + (py - ly) ** 2 + (pz - lz) ** 2
        d = jnp.minimum(d, dd)
        m = jnp.max(d, axis=1, keepdims=True)
        idx = jnp.min(jnp.where(d == m, lane, _N0), axis=1, keepdims=True)
        oh = (lane == idx).astype(jnp.float32)
        lx = jnp.sum(px * oh, axis=1, keepdims=True)
        ly = jnp.sum(py * oh, axis=1, keepdims=True)
        lz = jnp.sum(pz * oh, axis=1, keepdims=True)
        ohi = (lane2 == i).astype(jnp.float32)
        ax = ax + lx * ohi
        ay = ay + ly * ohi
        az = az + lz * ohi
        return (d, lx, ly, lz, ax, ay, az)

    st = jax.lax.fori_loop(1, _M2, body, (d, lx, ly, lz, ax, ay, az))
    _, _, _, _, ax, ay, az = st
    out_ref[0] = ax
    out_ref[1] = ay
    out_ref[2] = az


def _relu(v):
    return jnp.maximum(v, 0.0)


def _dot(a, b):
    return jnp.dot(a, b, preferred_element_type=jnp.float32)


def _cloud_body(x_ref, pos_ref, posT_ref, p2_ref, p2T_ref,
                w10_ref, b10_ref, w11_ref, b11_ref, w12_ref, b12_ref,
                w20_ref, b20_ref, w21_ref, b21_ref, w22_ref, b22_ref,
                w30_ref, b30_ref, w31_ref, b31_ref, w32_ref, b32_ref,
                wf30_ref, bf30_ref, wf31_ref, bf31_ref,
                wf20_ref, bf20_ref, wf21_ref, bf21_ref,
                wf10_ref, bf10_ref, wf11_ref, bf11_ref,
                out_ref,
                s_d2, s_a1, s_c1, s_t1, s_t3, s_x1,
                s_a2, s_c2, s_p2c, s_t2, s_x2):
    x = x_ref[0]
    pos = pos_ref[0]
    posT = posT_ref[0]
    p2 = p2_ref[0]
    p2T = p2T_ref[0]

    px = pos[:, 0:1]
    py = pos[:, 1:2]
    pz = pos[:, 2:3]
    pxr = posT[0:1, :]
    pyr = posT[1:2, :]
    pzr = posT[2:3, :]

    s_d2[...] = ((px - pxr) ** 2 + (py - pyr) ** 2) + (pz - pzr) ** 2

    t1f, t3f = _search2(
        lambda: jax.lax.bitcast_convert_type(s_d2[...], jnp.int32),
        _N0, _KNBR, 3)
    s_t1[...] = t1f
    s_t3[...] = t3f

    feat = jnp.concatenate([x, pos], axis=1)
    s_a1[...] = _dot(feat, w10_ref[...]) + b10_ref[...]
    s_c1[...] = _dot(pos, w10_ref[1:4, :])

    w11 = w11_ref[...]
    b11 = b11_ref[...]
    w12 = w12_ref[...]
    b12 = b12_ref[...]
    r1sq = jnp.float32(_R1 * _R1)

    def sa1_tile(t, _):
        base = t * _TC1
        a1 = s_a1[...]
        pieces = [a1 - s_c1[pl.ds(base + k, 1), :] for k in range(_TC1)]
        h = _relu(jnp.concatenate(pieces, axis=0))
        h = _relu(_dot(h, w11) + b11)
        h = _relu(_dot(h, w12) + b12)
        for k in range(_TC1):
            i = base + k
            c = pos_ref[0, pl.ds(i, 1), :]
            d2c = ((px - c[:, 0:1]) ** 2 + (py - c[:, 1:2]) ** 2) + (pz - c[:, 2:3]) ** 2
            tf = s_t1[pl.ds(i, 1), :]
            m = ((d2c <= tf) & (d2c <= r1sq)).astype(jnp.float32)
            blk = h[k * _N0:(k + 1) * _N0, :]
            s_x1[pl.ds(i, 1), :] = jnp.max(blk * m, axis=0, keepdims=True)
        return 0

    jax.lax.fori_loop(0, _N0 // _TC1, sa1_tile, 0)

    p2x = p2[:, 0:1]
    p2y = p2[:, 1:2]
    p2z = p2[:, 2:3]
    d2_21 = ((p2x - pxr) ** 2 + (p2y - pyr) ** 2) + (p2z - pzr) ** 2
    bits21 = jax.lax.bitcast_convert_type(d2_21, jnp.int32)
    t2f, _unused = _search2(lambda: bits21, _M2, _KNBR, 1)
    s_t2[...] = jnp.pad(t2f, ((0, _MP - _M2), (0, 0)), constant_values=-1.0)
    s_p2c[...] = jnp.pad(p2, ((0, _MP - _M2), (0, 0)))

    x1 = s_x1[...]
    feat2 = jnp.concatenate([x1, pos], axis=1)
    s_a2[...] = _dot(feat2, w20_ref[...]) + b20_ref[...]
    s_c2[...] = jnp.pad(_dot(p2, w20_ref[128:131, :]), ((0, _MP - _M2), (0, 0)))

    w21 = w21_ref[...]
    b21 = b21_ref[...]
    w22 = w22_ref[...]
    b22 = b22_ref[...]
    r2sq = jnp.float32(_R2 * _R2)

    def sa2_tile(t, _):
        base = t * _TC2
        a2 = s_a2[...]
        pieces = [a2 - s_c2[pl.ds(base + k, 1), :] for k in range(_TC2)]
        h = _relu(jnp.concatenate(pieces, axis=0))
        h = _relu(_dot(h, w21) + b21)
        h = _relu(_dot(h, w22) + b22)
        for k in range(_TC2):
            i = base + k
            c = s_p2c[pl.ds(i, 1), :]
            d2c = ((px - c[:, 0:1]) ** 2 + (py - c[:, 1:2]) ** 2) + (pz - c[:, 2:3]) ** 2
            tf = s_t2[pl.ds(i, 1), :]
            m = ((d2c <= tf) & (d2c <= r2sq)).astype(jnp.float32)
            blk = h[k * _N0:(k + 1) * _N0, :]
            s_x2[pl.ds(i, 1), :] = jnp.max(blk * m, axis=0, keepdims=True)
        return 0

    jax.lax.fori_loop(0, _MP // _TC2, sa2_tile, 0)

    x2 = s_x2[0:_M2, :]
    g = jnp.concatenate([x2, p2], axis=1)
    g = _relu(_dot(g, w30_ref[...]) + b30_ref[...])
    g = _relu(_dot(g, w31_ref[...]) + b31_ref[...])
    h3 = _relu(_dot(g, w32_ref[...]) + b32_ref[...])
    x3 = jnp.max(h3, axis=0, keepdims=True)

    xi3 = jnp.broadcast_to(x3, (_M2, 512))
    g = jnp.concatenate([xi3, x2], axis=1)
    g = _relu(_dot(g, wf30_ref[...]) + bf30_ref[...])
    f3 = _relu(_dot(g, wf31_ref[...]) + bf31_ref[...])

    p2xr = p2T[0:1, :]
    p2yr = p2T[1:2, :]
    p2zr = p2T[2:3, :]
    d2_12 = ((px - p2xr) ** 2 + (py - p2yr) ** 2) + (pz - p2zr) ** 2
    bits12 = jax.lax.bitcast_convert_type(d2_12, jnp.int32)
    tkf, _unused = _search2(lambda: bits12, _N0, 3, 1)
    w = jnp.where(d2_12 <= tkf, 1.0 / jnp.clip(d2_12, 1e-16, None), 0.0)
    xi2 = _dot(w, f3) / jnp.sum(w, axis=1, keepdims=True)
    g = jnp.concatenate([xi2, x1], axis=1)
    g = _relu(_dot(g, wf20_ref[...]) + bf20_ref[...])
    f2 = _relu(_dot(g, wf21_ref[...]) + bf21_ref[...])

    d2 = s_d2[...]
    w = jnp.where(d2 <= s_t3[...], 1.0 / jnp.clip(d2, 1e-16, None), 0.0)
    xi1 = _dot(w, f2) / jnp.sum(w, axis=1, keepdims=True)
    g = jnp.concatenate([xi1, x], axis=1)
    g = _relu(_dot(g, wf10_ref[...]) + bf10_ref[...])
    f1 = _relu(_dot(g, wf11_ref[...]) + bf11_ref[...])
    out_ref[0] = f1


def _full(shape):
    return pl.BlockSpec(shape, lambda b: (0,) * len(shape))


def kernel(x, pos, batch, params):
    xb = x.reshape(_B, _N0, 1)
    pb = pos.reshape(_B, _N0, 3)
    pbT3 = pb.transpose(2, 0, 1)

    p2_3 = pl.pallas_call(
        _fps_body,
        out_shape=jax.ShapeDtypeStruct((3, _B, _M2), jnp.float32),
    )(pbT3)
    p2 = p2_3.transpose(1, 2, 0)
    p2T = p2_3.transpose(1, 0, 2)
    pbT = pbT3.transpose(1, 0, 2)

    wargs = []
    specs = []

    def addw(a):
        a = jnp.asarray(a, jnp.float32)
        if a.ndim == 1:
            a = a[None, :]
        wargs.append(a)
        specs.append(_full(a.shape))

    for name in ('sa1', 'sa2', 'sa3', 'fp3', 'fp2', 'fp1'):
        for Wb in params[name]:
            addw(Wb[0])
            addw(Wb[1])

    in_specs = [
        pl.BlockSpec((1, _N0, 1), lambda b: (b, 0, 0)),
        pl.BlockSpec((1, _N0, 3), lambda b: (b, 0, 0)),
        pl.BlockSpec((1, 3, _N0), lambda b: (b, 0, 0)),
        pl.BlockSpec((1, _M2, 3), lambda b: (b, 0, 0)),
        pl.BlockSpec((1, 3, _M2), lambda b: (b, 0, 0)),
    ] + specs

    out = pl.pallas_call(
        _cloud_body,
        grid=(_B,),
        in_specs=in_specs,
        out_specs=pl.BlockSpec((1, _N0, 128), lambda b: (b, 0, 0)),
        out_shape=jax.ShapeDtypeStruct((_B, _N0, 128), jnp.float32),
        scratch_shapes=[
            pltpu.VMEM((_N0, _N0), jnp.float32),
            pltpu.VMEM((_N0, 64), jnp.float32),
            pltpu.VMEM((_N0, 64), jnp.float32),
            pltpu.VMEM((_N0, 1), jnp.float32),
            pltpu.VMEM((_N0, 1), jnp.float32),
            pltpu.VMEM((_N0, 128), jnp.float32),
            pltpu.VMEM((_N0, 128), jnp.float32),
            pltpu.VMEM((_MP, 128), jnp.float32),
            pltpu.VMEM((_MP, 3), jnp.float32),
            pltpu.VMEM((_MP, 1), jnp.float32),
            pltpu.VMEM((_MP, 256), jnp.float32),
        ],
    )(xb, pb, pbT, p2, p2T, *wargs)
    return out.reshape(_B * _N0, 128)
